# baseline (speedup 1.0000x reference)
import numpy as np
import jax
import jax.numpy as jnp
from jax.experimental import pallas as pl
from jax.experimental.pallas import tpu as pltpu

_T, _D, _E = 32768, 768, 64
_BLK = 1024


def _noise_block_t(base_u32):
    shape = (_E, _BLK)
    sub = jax.lax.broadcasted_iota(jnp.uint32, shape, 0)
    lane = jax.lax.broadcasted_iota(jnp.uint32, shape, 1)
    x1 = base_u32 + lane * jnp.uint32(_E) + sub
    x0 = jnp.zeros(shape, jnp.uint32)
    ks = (jnp.uint32(0), jnp.uint32(42),
          jnp.uint32(0) ^ jnp.uint32(42) ^ jnp.uint32(0x1BD11BDA))
    rot = ((13, 15, 26, 6), (17, 29, 16, 24))
    x0 = x0 + ks[0]
    x1 = x1 + ks[1]
    for i in range(5):
        for r in rot[i % 2]:
            x0 = x0 + x1
            x1 = (x1 << jnp.uint32(r)) | (x1 >> jnp.uint32(32 - r))
            x1 = x0 ^ x1
        x0 = x0 + ks[(i + 1) % 3]
        x1 = x1 + ks[(i + 2) % 3] + jnp.uint32(i + 1)
    bits = x0 ^ x1
    fl = jax.lax.bitcast_convert_type(
        (bits >> jnp.uint32(9)) | jnp.uint32(0x3F800000), jnp.float32) - 1.0
    lo = jnp.float32(np.nextafter(np.float32(-1.0), np.float32(0.0)))
    u = jnp.maximum(lo, fl * (jnp.float32(1.0) - lo) + lo)
    return jnp.float32(np.sqrt(2.0)) * jax.lax.erf_inv(u)


def _router_block(x_ref, gw_ref, gb_ref, nw_ref, nb_ref, probs_ref, idx_ref):
    x = x_ref[...]
    dn = (((1,), (1,)), ((), ()))
    logits = jax.lax.dot_general(
        gw_ref[...], x, dn, preferred_element_type=jnp.float32) + gb_ref[...][:, None]
    nlog = jax.lax.dot_general(
        nw_ref[...], x, dn, preferred_element_type=jnp.float32) + nb_ref[...][:, None]
    std = jax.nn.softplus(nlog)
    nz = _noise_block_t(pl.program_id(0).astype(jnp.uint32) * jnp.uint32(_BLK * _E))
    noisy = logits + nz * std
    maxv = jnp.max(noisy, axis=0, keepdims=True)
    sub = jax.lax.broadcasted_iota(jnp.int32, noisy.shape, 0)
    idx = jnp.min(jnp.where(noisy == maxv, sub, _E), axis=0, keepdims=True)
    probs_ref[...] = (sub == idx).astype(jnp.float32)
    idx_ref[...] = idx


def kernel(x, gate_w, gate_b, noise_w, noise_b):
    probs_t, idx_t = pl.pallas_call(
        _router_block,
        grid=(_T // _BLK,),
        compiler_params=pltpu.CompilerParams(
            dimension_semantics=("arbitrary",)),
        in_specs=[
            pl.BlockSpec((_BLK, _D), lambda i: (i, 0)),
            pl.BlockSpec((_E, _D), lambda i: (0, 0)),
            pl.BlockSpec((_E,), lambda i: (0,)),
            pl.BlockSpec((_E, _D), lambda i: (0, 0)),
            pl.BlockSpec((_E,), lambda i: (0,)),
        ],
        out_specs=[
            pl.BlockSpec((_E, _BLK), lambda i: (0, i)),
            pl.BlockSpec((1, _BLK), lambda i: (0, i)),
        ],
        out_shape=[
            jax.ShapeDtypeStruct((_E, _T), jnp.float32),
            jax.ShapeDtypeStruct((1, _T), jnp.int32),
        ],
    )(x, gate_w, gate_b, noise_w, noise_b)
    return probs_t.T, idx_t.T


# speedup vs baseline: 1.0600x; 1.0600x over previous
import numpy as np
import jax
import jax.numpy as jnp
from jax.experimental import pallas as pl
from jax.experimental.pallas import tpu as pltpu

_T, _D, _E = 32768, 768, 64
_BLK = 2048


def _noise_block_t(base_u32):
    shape = (_E, _BLK)
    sub = jax.lax.broadcasted_iota(jnp.uint32, shape, 0)
    lane = jax.lax.broadcasted_iota(jnp.uint32, shape, 1)
    x1 = base_u32 + lane * jnp.uint32(_E) + sub
    x0 = jnp.zeros(shape, jnp.uint32)
    ks = (jnp.uint32(0), jnp.uint32(42),
          jnp.uint32(0) ^ jnp.uint32(42) ^ jnp.uint32(0x1BD11BDA))
    rot = ((13, 15, 26, 6), (17, 29, 16, 24))
    x0 = x0 + ks[0]
    x1 = x1 + ks[1]
    for i in range(5):
        for r in rot[i % 2]:
            x0 = x0 + x1
            x1 = (x1 << jnp.uint32(r)) | (x1 >> jnp.uint32(32 - r))
            x1 = x0 ^ x1
        x0 = x0 + ks[(i + 1) % 3]
        x1 = x1 + ks[(i + 2) % 3] + jnp.uint32(i + 1)
    bits = x0 ^ x1
    fl = jax.lax.bitcast_convert_type(
        (bits >> jnp.uint32(9)) | jnp.uint32(0x3F800000), jnp.float32) - 1.0
    lo = jnp.float32(np.nextafter(np.float32(-1.0), np.float32(0.0)))
    u = jnp.maximum(lo, fl * (jnp.float32(1.0) - lo) + lo)
    return jnp.float32(np.sqrt(2.0)) * jax.lax.erf_inv(u)


def _router_block(x_ref, gw_ref, gb_ref, nw_ref, nb_ref, probs_ref, idx_ref):
    x = x_ref[...]
    dn = (((1,), (1,)), ((), ()))
    logits = jax.lax.dot_general(
        gw_ref[...], x, dn, preferred_element_type=jnp.float32) + gb_ref[...][:, None]
    nlog = jax.lax.dot_general(
        nw_ref[...], x, dn, preferred_element_type=jnp.float32) + nb_ref[...][:, None]
    std = jax.nn.softplus(nlog)
    nz = _noise_block_t(pl.program_id(0).astype(jnp.uint32) * jnp.uint32(_BLK * _E))
    noisy = logits + nz * std
    maxv = jnp.max(noisy, axis=0, keepdims=True)
    sub = jax.lax.broadcasted_iota(jnp.int32, noisy.shape, 0)
    idx = jnp.min(jnp.where(noisy == maxv, sub, _E), axis=0, keepdims=True)
    probs_ref[...] = (sub == idx).astype(jnp.float32)
    idx_ref[...] = idx


def kernel(x, gate_w, gate_b, noise_w, noise_b):
    probs_t, idx_t = pl.pallas_call(
        _router_block,
        grid=(_T // _BLK,),
        compiler_params=pltpu.CompilerParams(
            dimension_semantics=("arbitrary",)),
        in_specs=[
            pl.BlockSpec((_BLK, _D), lambda i: (i, 0)),
            pl.BlockSpec((_E, _D), lambda i: (0, 0)),
            pl.BlockSpec((_E,), lambda i: (0,)),
            pl.BlockSpec((_E, _D), lambda i: (0, 0)),
            pl.BlockSpec((_E,), lambda i: (0,)),
        ],
        out_specs=[
            pl.BlockSpec((_E, _BLK), lambda i: (0, i)),
            pl.BlockSpec((1, _BLK), lambda i: (0, i)),
        ],
        out_shape=[
            jax.ShapeDtypeStruct((_E, _T), jnp.float32),
            jax.ShapeDtypeStruct((1, _T), jnp.int32),
        ],
    )(x, gate_w, gate_b, noise_w, noise_b)
    return probs_t.T, idx_t.T
